# Initial kernel scaffold; baseline (speedup 1.0000x reference)
import jax
import jax.numpy as jnp
from jax.experimental import pallas as pl


def kernel(x, edge_index, table, W1, b1, W2, b2, W3, b3, Wd, bd):
    raise NotImplementedError("write your pallas kernel here")


# speedup vs baseline: 8.7753x; 8.7753x over previous
import functools

import jax
import jax.numpy as jnp
from jax import lax
from jax.experimental import pallas as pl
from jax.experimental.pallas import tpu as pltpu
from jax.experimental.pallas import tpu_sc as plsc

N = 2048
VOCAB = 2048
HID = 128
MID = 64
E = 32768
NC, NS = 2, 16
NW = NC * NS
EPW = E // NW
CHUNK = 128
NCHUNK = EPW // CHUNK
RPW = N // NW


def _sc_gather_deg_body(table_hbm, x_hbm, dst_hbm, ones_hbm, zeros_hbm,
                        feats_out, deg_out,
                        xidx_v, rows_v, didx_v, ones_v, sem, deg_sh):
    c = lax.axis_index("c")
    s = lax.axis_index("s")
    wid = s * NC + c

    pltpu.sync_copy(zeros_hbm, deg_sh.at[pl.ds(s * CHUNK, CHUNK)])
    fbase = wid * RPW
    pltpu.sync_copy(x_hbm.at[pl.ds(fbase, RPW)], xidx_v)
    pltpu.async_copy(table_hbm.at[xidx_v], rows_v, sem).wait()
    pltpu.sync_copy(rows_v, feats_out.at[pl.ds(fbase, RPW)])

    pltpu.sync_copy(ones_hbm, ones_v)
    plsc.subcore_barrier()
    for i in range(NCHUNK):
        base = wid * EPW + i * CHUNK
        pltpu.sync_copy(dst_hbm.at[pl.ds(base, CHUNK)], didx_v)
        pltpu.sync_copy(ones_v, deg_sh.at[didx_v], add=True)
    plsc.subcore_barrier()
    pltpu.sync_copy(deg_sh.at[pl.ds(s * CHUNK, CHUNK)],
                    deg_out.at[pl.ds(c * N + s * CHUNK, CHUNK)])


@functools.cache
def _get_sc_gather_deg():
    mesh = plsc.VectorSubcoreMesh(core_axis_name="c", subcore_axis_name="s",
                                  num_cores=NC, num_subcores=NS)
    return pl.kernel(
        _sc_gather_deg_body,
        out_type=[
            jax.ShapeDtypeStruct((N, HID), jnp.float32),
            jax.ShapeDtypeStruct((NC * N, HID), jnp.float32),
        ],
        mesh=mesh,
        scratch_types=[
            pltpu.VMEM((RPW,), jnp.int32),
            pltpu.VMEM((RPW, HID), jnp.float32),
            pltpu.VMEM((CHUNK,), jnp.int32),
            pltpu.VMEM((CHUNK, HID), jnp.float32),
            pltpu.SemaphoreType.DMA,
            pltpu.VMEM_SHARED((N, HID), jnp.float32),
        ],
    )


def _sc_scatter_body(q_hbm, src_hbm, dst_hbm, zeros_hbm,
                     s_out,
                     sidx_v, didx_v, rows_v, sem, agg_sh):
    c = lax.axis_index("c")
    s = lax.axis_index("s")
    wid = s * NC + c

    pltpu.sync_copy(zeros_hbm, agg_sh.at[pl.ds(s * CHUNK, CHUNK)])
    plsc.subcore_barrier()
    for i in range(NCHUNK):
        base = wid * EPW + i * CHUNK
        pltpu.sync_copy(src_hbm.at[pl.ds(base, CHUNK)], sidx_v)
        pltpu.async_copy(q_hbm.at[sidx_v], rows_v, sem).wait()
        pltpu.sync_copy(dst_hbm.at[pl.ds(base, CHUNK)], didx_v)
        pltpu.sync_copy(rows_v, agg_sh.at[didx_v], add=True)
    plsc.subcore_barrier()
    pltpu.sync_copy(agg_sh.at[pl.ds(s * CHUNK, CHUNK)],
                    s_out.at[pl.ds(c * N + s * CHUNK, CHUNK)])


@functools.cache
def _get_sc_scatter():
    mesh = plsc.VectorSubcoreMesh(core_axis_name="c", subcore_axis_name="s",
                                  num_cores=NC, num_subcores=NS)
    return pl.kernel(
        _sc_scatter_body,
        out_type=jax.ShapeDtypeStruct((NC * N, HID), jnp.float32),
        mesh=mesh,
        scratch_types=[
            pltpu.VMEM((CHUNK,), jnp.int32),
            pltpu.VMEM((CHUNK,), jnp.int32),
            pltpu.VMEM((CHUNK, HID), jnp.float32),
            pltpu.SemaphoreType.DMA,
            pltpu.VMEM_SHARED((N, HID), jnp.float32),
        ],
    )


def _tc_a_body(deg_ref, feats_ref, w1_ref, q1_ref, dinv_ref):
    deg = deg_ref[0:N, 0:1] + deg_ref[N:2 * N, 0:1] + 1.0
    dinv = lax.rsqrt(deg)
    dinv_ref[...] = dinv
    p1 = jnp.dot(feats_ref[...], w1_ref[...],
                 preferred_element_type=jnp.float32)
    q1_ref[:, 0:MID] = dinv * p1
    q1_ref[:, MID:HID] = jnp.zeros((N, HID - MID), jnp.float32)


def _tc_b_body(s1_ref, q1_ref, dinv_ref, b1_ref, w2_ref, q2_ref):
    dinv = dinv_ref[...]
    q1 = q1_ref[:, 0:MID]
    s1 = s1_ref[0:N, 0:MID] + s1_ref[N:2 * N, 0:MID]
    h1 = jnp.maximum(dinv * (s1 + q1) + b1_ref[...], 0.0)
    p2 = jnp.dot(h1, w2_ref[...], preferred_element_type=jnp.float32)
    q2_ref[:, 0:MID] = dinv * p2
    q2_ref[:, MID:HID] = jnp.zeros((N, HID - MID), jnp.float32)


def _tc_c1_body(s2_ref, q2_ref, dinv_ref, b2_ref, w3_ref, b3_ref,
                wd_ref, bd_ref, h3_ref, r_ref, t_ref, tt_ref):
    dinv = dinv_ref[...]
    q2 = q2_ref[:, 0:MID]
    s2 = s2_ref[0:N, 0:MID] + s2_ref[N:2 * N, 0:MID]
    h2 = jnp.maximum(dinv * (s2 + q2) + b2_ref[...], 0.0)
    h3 = jnp.dot(h2, w3_ref[...], preferred_element_type=jnp.float32)
    h3 = h3 + b3_ref[...]
    h3_ref[...] = h3
    wd = wd_ref[...]
    g = lax.dot_general(wd, wd, (((1,), (1,)), ((), ())),
                        preferred_element_type=jnp.float32)
    bd = bd_ref[...]
    u = lax.dot_general(wd, bd, (((1,), (1,)), ((), ())),
                        preferred_element_type=jnp.float32)
    c = jnp.sum(bd * bd)
    r_ref[...] = jnp.dot(h3, g, preferred_element_type=jnp.float32)
    t = jnp.dot(h3, u, preferred_element_type=jnp.float32) + 0.5 * c
    t_ref[...] = t
    tt_ref[...] = t.reshape(1, N)


_BM = 512


def _tc_c2_body(r_ref, h3_ref, t_ref, tt_ref, o_ref):
    acc = lax.dot_general(r_ref[...], h3_ref[...], (((1,), (1,)), ((), ())),
                          preferred_element_type=jnp.float32)
    o_ref[...] = jax.nn.sigmoid(acc + t_ref[...] + tt_ref[...])


@functools.cache
def _get_tc_c2():
    return pl.pallas_call(
        _tc_c2_body,
        grid=(N // _BM, N // _BM),
        in_specs=[
            pl.BlockSpec((_BM, HID), lambda i, j: (i, 0)),
            pl.BlockSpec((_BM, HID), lambda i, j: (j, 0)),
            pl.BlockSpec((_BM, 1), lambda i, j: (i, 0)),
            pl.BlockSpec((1, _BM), lambda i, j: (0, j)),
        ],
        out_specs=pl.BlockSpec((_BM, _BM), lambda i, j: (i, j)),
        out_shape=jax.ShapeDtypeStruct((N, N), jnp.float32),
    )


def kernel(x, edge_index, table, W1, b1, W2, b2, W3, b3, Wd, bd):
    src = edge_index[0]
    dst = edge_index[1]
    ones128 = jnp.ones((CHUNK, HID), jnp.float32)
    zeros128 = jnp.zeros((CHUNK, HID), jnp.float32)

    feats, deg_parts = _get_sc_gather_deg()(table, x, dst, ones128, zeros128)

    q1, dinv = pl.pallas_call(
        _tc_a_body,
        out_shape=[
            jax.ShapeDtypeStruct((N, HID), jnp.float32),
            jax.ShapeDtypeStruct((N, 1), jnp.float32),
        ],
    )(deg_parts, feats, W1)

    s1_parts = _get_sc_scatter()(q1, src, dst, zeros128)

    q2 = pl.pallas_call(
        _tc_b_body,
        out_shape=jax.ShapeDtypeStruct((N, HID), jnp.float32),
    )(s1_parts, q1, dinv, b1.reshape(1, MID), W2)

    s2_parts = _get_sc_scatter()(q2, src, dst, zeros128)

    h3, r, t, tt = pl.pallas_call(
        _tc_c1_body,
        out_shape=[
            jax.ShapeDtypeStruct((N, HID), jnp.float32),
            jax.ShapeDtypeStruct((N, HID), jnp.float32),
            jax.ShapeDtypeStruct((N, 1), jnp.float32),
            jax.ShapeDtypeStruct((1, N), jnp.float32),
        ],
    )(s2_parts, q2, dinv, b2.reshape(1, MID), W3, b3.reshape(1, HID),
      Wd, bd.reshape(1, VOCAB))

    return _get_tc_c2()(r, h3, t, tt)


# speedup vs baseline: 10.6052x; 1.2085x over previous
import functools

import jax
import jax.numpy as jnp
from jax import lax
from jax.experimental import pallas as pl
from jax.experimental.pallas import tpu as pltpu
from jax.experimental.pallas import tpu_sc as plsc

N = 2048
VOCAB = 2048
HID = 128
MID = 64
E = 32768
NC, NS = 2, 16
NW = NC * NS
EPW = E // NW
CHUNK = 128
NCHUNK = EPW // CHUNK
RPW = N // NW


def _sc_gather_deg_body(table_hbm, x_hbm, dst3d_hbm, ones_hbm, zeros_hbm,
                        feats_out, deg_out,
                        xidx_v, rows_v, didx_0, didx_1, ones_v, gsem, ssem,
                        dsems, deg_sh):
    c = lax.axis_index("c")
    s = lax.axis_index("s")
    wid = s * NC + c
    dbufs = [didx_0, didx_1]

    pltpu.sync_copy(zeros_hbm, deg_sh.at[pl.ds(s * CHUNK, CHUNK)])
    pltpu.sync_copy(ones_hbm, ones_v)
    fbase = wid * RPW
    pltpu.sync_copy(x_hbm.at[pl.ds(fbase, RPW)], xidx_v)
    fg = pltpu.async_copy(table_hbm.at[xidx_v], rows_v, gsem)
    plsc.subcore_barrier()
    for i in range(NCHUNK):
        pltpu.sync_copy(dst3d_hbm.at[wid, i], didx_0)
        pltpu.sync_copy(ones_v, deg_sh.at[didx_0], add=True)
    fg.wait()
    pltpu.sync_copy(rows_v, feats_out.at[pl.ds(fbase, RPW)])
    plsc.subcore_barrier()
    pltpu.sync_copy(deg_sh.at[pl.ds(s * CHUNK, CHUNK)],
                    deg_out.at[pl.ds(c * N + s * CHUNK, CHUNK)])


@functools.cache
def _get_sc_gather_deg():
    mesh = plsc.VectorSubcoreMesh(core_axis_name="c", subcore_axis_name="s",
                                  num_cores=NC, num_subcores=NS)
    return pl.kernel(
        _sc_gather_deg_body,
        out_type=[
            jax.ShapeDtypeStruct((N, HID), jnp.float32),
            jax.ShapeDtypeStruct((NC * N, HID), jnp.float32),
        ],
        mesh=mesh,
        scratch_types=[
            pltpu.VMEM((RPW,), jnp.int32),
            pltpu.VMEM((RPW, HID), jnp.float32),
            pltpu.VMEM((CHUNK,), jnp.int32),
            pltpu.VMEM((CHUNK,), jnp.int32),
            pltpu.VMEM((CHUNK, HID), jnp.float32),
            pltpu.SemaphoreType.DMA,
            pltpu.SemaphoreType.DMA,
            pltpu.SemaphoreType.DMA((2,)),
            pltpu.VMEM_SHARED((N, HID), jnp.float32),
        ],
    )


NBUF = 4


def _sc_scatter_body(q_hbm, src3d_hbm, dst3d_hbm, zeros_hbm,
                     s_out,
                     sidx_bufs, didx_bufs, row_bufs, gsems, ssems, dsems,
                     isems, agg_sh):
    c = lax.axis_index("c")
    s = lax.axis_index("s")
    wid = s * NC + c

    pltpu.sync_copy(zeros_hbm, agg_sh.at[pl.ds(s * CHUNK, CHUNK)])
    icp = [pltpu.async_copy(src3d_hbm.at[wid, i], sidx_bufs[i],
                            isems.at[i]) for i in range(NCHUNK)]
    plsc.subcore_barrier()
    gathers = [None] * NCHUNK
    for b in range(NBUF):
        icp[b].wait()
        gathers[b] = pltpu.async_copy(
            q_hbm.at[sidx_bufs[b]], row_bufs[b], gsems.at[b])
    for i in range(NCHUNK):
        b = i % NBUF
        gathers[i].wait()
        pltpu.sync_copy(dst3d_hbm.at[wid, i], didx_bufs[0])
        pltpu.sync_copy(row_bufs[b], agg_sh.at[didx_bufs[0]], add=True)
        if i + NBUF < NCHUNK:
            icp[i + NBUF].wait()
            gathers[i + NBUF] = pltpu.async_copy(
                q_hbm.at[sidx_bufs[i + NBUF]], row_bufs[b], gsems.at[b])
    plsc.subcore_barrier()
    pltpu.sync_copy(agg_sh.at[pl.ds(s * CHUNK, CHUNK)],
                    s_out.at[pl.ds(c * N + s * CHUNK, CHUNK)])


@functools.cache
def _get_sc_scatter():
    mesh = plsc.VectorSubcoreMesh(core_axis_name="c", subcore_axis_name="s",
                                  num_cores=NC, num_subcores=NS)
    return pl.kernel(
        _sc_scatter_body,
        out_type=jax.ShapeDtypeStruct((NC * N, HID), jnp.float32),
        mesh=mesh,
        scratch_types=[
            [pltpu.VMEM((CHUNK,), jnp.int32) for _ in range(NCHUNK)],
            [pltpu.VMEM((CHUNK,), jnp.int32) for _ in range(2)],
            [pltpu.VMEM((CHUNK, HID), jnp.float32) for _ in range(NBUF)],
            pltpu.SemaphoreType.DMA((NBUF,)),
            pltpu.SemaphoreType.DMA((NBUF,)),
            pltpu.SemaphoreType.DMA((2,)),
            pltpu.SemaphoreType.DMA((NCHUNK,)),
            pltpu.VMEM_SHARED((N, HID), jnp.float32),
        ],
    )


def _tc_a_body(deg_ref, feats_ref, w1_ref, q1_ref, dinv_ref):
    deg = deg_ref[0:N, 0:1] + deg_ref[N:2 * N, 0:1] + 1.0
    dinv = lax.rsqrt(deg)
    dinv_ref[...] = dinv
    p1 = jnp.dot(feats_ref[...], w1_ref[...],
                 preferred_element_type=jnp.float32)
    q1_ref[:, 0:MID] = dinv * p1
    q1_ref[:, MID:HID] = jnp.zeros((N, HID - MID), jnp.float32)


def _tc_b_body(s1_ref, q1_ref, dinv_ref, b1_ref, w2_ref, q2_ref):
    dinv = dinv_ref[...]
    q1 = q1_ref[:, 0:MID]
    s1 = s1_ref[0:N, 0:MID] + s1_ref[N:2 * N, 0:MID]
    h1 = jnp.maximum(dinv * (s1 + q1) + b1_ref[...], 0.0)
    p2 = jnp.dot(h1, w2_ref[...], preferred_element_type=jnp.float32)
    q2_ref[:, 0:MID] = dinv * p2
    q2_ref[:, MID:HID] = jnp.zeros((N, HID - MID), jnp.float32)


def _tc_c1_body(s2_ref, q2_ref, dinv_ref, b2_ref, w3_ref, b3_ref,
                wd_ref, bd_ref, h3_ref, r_ref, t_ref, tt_ref):
    dinv = dinv_ref[...]
    q2 = q2_ref[:, 0:MID]
    s2 = s2_ref[0:N, 0:MID] + s2_ref[N:2 * N, 0:MID]
    h2 = jnp.maximum(dinv * (s2 + q2) + b2_ref[...], 0.0)
    h3 = jnp.dot(h2, w3_ref[...], preferred_element_type=jnp.float32)
    h3 = h3 + b3_ref[...]
    h3_ref[...] = h3
    wd = wd_ref[...]
    g = lax.dot_general(wd, wd, (((1,), (1,)), ((), ())),
                        preferred_element_type=jnp.float32)
    bd = bd_ref[...]
    u = lax.dot_general(wd, bd, (((1,), (1,)), ((), ())),
                        preferred_element_type=jnp.float32)
    c = jnp.sum(bd * bd)
    r_ref[...] = jnp.dot(h3, g, preferred_element_type=jnp.float32)
    t = jnp.dot(h3, u, preferred_element_type=jnp.float32) + 0.5 * c
    t_ref[...] = t
    tt_ref[...] = t.reshape(1, N)


_BM = 512


def _tc_c2_body(r_ref, h3_ref, t_ref, tt_ref, o_ref):
    acc = lax.dot_general(r_ref[...], h3_ref[...], (((1,), (1,)), ((), ())),
                          preferred_element_type=jnp.float32)
    o_ref[...] = jax.nn.sigmoid(acc + t_ref[...] + tt_ref[...])


@functools.cache
def _get_tc_c2():
    return pl.pallas_call(
        _tc_c2_body,
        grid=(N // _BM, N // _BM),
        in_specs=[
            pl.BlockSpec((_BM, HID), lambda i, j: (i, 0)),
            pl.BlockSpec((_BM, HID), lambda i, j: (j, 0)),
            pl.BlockSpec((_BM, 1), lambda i, j: (i, 0)),
            pl.BlockSpec((1, _BM), lambda i, j: (0, j)),
        ],
        out_specs=pl.BlockSpec((_BM, _BM), lambda i, j: (i, j)),
        out_shape=jax.ShapeDtypeStruct((N, N), jnp.float32),
    )


def kernel(x, edge_index, table, W1, b1, W2, b2, W3, b3, Wd, bd):
    src3d = edge_index[0].reshape(NW, NCHUNK, CHUNK)
    dst3d = edge_index[1].reshape(NW, NCHUNK, CHUNK)
    ones128 = jnp.ones((CHUNK, HID), jnp.float32)
    zeros128 = jnp.zeros((CHUNK, HID), jnp.float32)

    feats, deg_parts = _get_sc_gather_deg()(table, x, dst3d, ones128, zeros128)

    q1, dinv = pl.pallas_call(
        _tc_a_body,
        out_shape=[
            jax.ShapeDtypeStruct((N, HID), jnp.float32),
            jax.ShapeDtypeStruct((N, 1), jnp.float32),
        ],
    )(deg_parts, feats, W1)

    s1_parts = _get_sc_scatter()(q1, src3d, dst3d, zeros128)

    q2 = pl.pallas_call(
        _tc_b_body,
        out_shape=jax.ShapeDtypeStruct((N, HID), jnp.float32),
    )(s1_parts, q1, dinv, b1.reshape(1, MID), W2)

    s2_parts = _get_sc_scatter()(q2, src3d, dst3d, zeros128)

    h3, r, t, tt = pl.pallas_call(
        _tc_c1_body,
        out_shape=[
            jax.ShapeDtypeStruct((N, HID), jnp.float32),
            jax.ShapeDtypeStruct((N, HID), jnp.float32),
            jax.ShapeDtypeStruct((N, 1), jnp.float32),
            jax.ShapeDtypeStruct((1, N), jnp.float32),
        ],
    )(s2_parts, q2, dinv, b2.reshape(1, MID), W3, b3.reshape(1, HID),
      Wd, bd.reshape(1, VOCAB))

    return _get_tc_c2()(r, h3, t, tt)


# speedup vs baseline: 11.0368x; 1.0407x over previous
import functools

import jax
import jax.numpy as jnp
from jax import lax
from jax.experimental import pallas as pl
from jax.experimental.pallas import tpu as pltpu
from jax.experimental.pallas import tpu_sc as plsc

N = 2048
VOCAB = 2048
HID = 128
MID = 64
E = 32768
NC, NS = 2, 16
NW = NC * NS
EPW = E // NW
CHUNK = 128
NCHUNK = EPW // CHUNK
RPW = N // NW


def _sc_gather_deg_body(table_hbm, x_hbm, dst3d_hbm, ones_hbm, zeros_hbm,
                        feats_out, deg_out,
                        xidx_v, rows_v, didx_0, didx_1, ones_v, gsem, ssem,
                        dsems, deg_sh):
    c = lax.axis_index("c")
    s = lax.axis_index("s")
    wid = s * NC + c
    dbufs = [didx_0, didx_1]

    pltpu.sync_copy(zeros_hbm, deg_sh.at[pl.ds(s * CHUNK, CHUNK)])
    pltpu.sync_copy(ones_hbm, ones_v)
    fbase = wid * RPW
    pltpu.sync_copy(x_hbm.at[pl.ds(fbase, RPW)], xidx_v)
    fg = pltpu.async_copy(table_hbm.at[xidx_v], rows_v, gsem)
    plsc.subcore_barrier()
    for i in range(NCHUNK):
        pltpu.sync_copy(dst3d_hbm.at[wid, i], dbufs[i % 2])
        pltpu.sync_copy(ones_v, deg_sh.at[dbufs[i % 2]], add=True)
    fg.wait()
    pltpu.sync_copy(rows_v, feats_out.at[pl.ds(fbase, RPW)])
    plsc.subcore_barrier()
    pltpu.sync_copy(deg_sh.at[pl.ds(s * CHUNK, CHUNK)],
                    deg_out.at[pl.ds(c * N + s * CHUNK, CHUNK)])


@functools.cache
def _get_sc_gather_deg():
    mesh = plsc.VectorSubcoreMesh(core_axis_name="c", subcore_axis_name="s",
                                  num_cores=NC, num_subcores=NS)
    return pl.kernel(
        _sc_gather_deg_body,
        out_type=[
            jax.ShapeDtypeStruct((N, HID), jnp.float32),
            jax.ShapeDtypeStruct((NC * N, 16), jnp.float32),
        ],
        mesh=mesh,
        scratch_types=[
            pltpu.VMEM((RPW,), jnp.int32),
            pltpu.VMEM((RPW, HID), jnp.float32),
            pltpu.VMEM((CHUNK,), jnp.int32),
            pltpu.VMEM((CHUNK,), jnp.int32),
            pltpu.VMEM((CHUNK, 16), jnp.float32),
            pltpu.SemaphoreType.DMA,
            pltpu.SemaphoreType.DMA,
            pltpu.SemaphoreType.DMA((2,)),
            pltpu.VMEM_SHARED((N, 16), jnp.float32),
        ],
        compiler_params=pltpu.CompilerParams(use_tc_tiling_on_sc=False),
    )


NBUF = 4


def _sc_scatter_body(q_hbm, src3d_hbm, dst3d_hbm, zeros_hbm,
                     s_out,
                     sidx_bufs, didx_bufs, row_bufs, gsems, ssems, dsems,
                     isems, agg_sh):
    c = lax.axis_index("c")
    s = lax.axis_index("s")
    wid = s * NC + c

    pltpu.sync_copy(zeros_hbm, agg_sh.at[pl.ds(s * CHUNK, CHUNK)])
    icp = [pltpu.async_copy(src3d_hbm.at[wid, i], sidx_bufs[i],
                            isems.at[i]) for i in range(NCHUNK)]
    plsc.subcore_barrier()
    gathers = [None] * NCHUNK
    for b in range(NBUF):
        icp[b].wait()
        gathers[b] = pltpu.async_copy(
            q_hbm.at[sidx_bufs[b]], row_bufs[b], gsems.at[b])
    for i in range(NCHUNK):
        b = i % NBUF
        gathers[i].wait()
        pltpu.sync_copy(dst3d_hbm.at[wid, i], didx_bufs[i % 2])
        pltpu.sync_copy(row_bufs[b], agg_sh.at[didx_bufs[i % 2]], add=True)
        if i + NBUF < NCHUNK:
            icp[i + NBUF].wait()
            gathers[i + NBUF] = pltpu.async_copy(
                q_hbm.at[sidx_bufs[i + NBUF]], row_bufs[b], gsems.at[b])
    plsc.subcore_barrier()
    pltpu.sync_copy(agg_sh.at[pl.ds(s * CHUNK, CHUNK)],
                    s_out.at[pl.ds(c * N + s * CHUNK, CHUNK)])


@functools.cache
def _get_sc_scatter():
    mesh = plsc.VectorSubcoreMesh(core_axis_name="c", subcore_axis_name="s",
                                  num_cores=NC, num_subcores=NS)
    return pl.kernel(
        _sc_scatter_body,
        out_type=jax.ShapeDtypeStruct((NC * N, HID), jnp.float32),
        mesh=mesh,
        scratch_types=[
            [pltpu.VMEM((CHUNK,), jnp.int32) for _ in range(NCHUNK)],
            [pltpu.VMEM((CHUNK,), jnp.int32) for _ in range(2)],
            [pltpu.VMEM((CHUNK, HID), jnp.float32) for _ in range(NBUF)],
            pltpu.SemaphoreType.DMA((NBUF,)),
            pltpu.SemaphoreType.DMA((NBUF,)),
            pltpu.SemaphoreType.DMA((2,)),
            pltpu.SemaphoreType.DMA((NCHUNK,)),
            pltpu.VMEM_SHARED((N, HID), jnp.float32),
        ],
    )


def _tc_a_body(deg_ref, feats_ref, w1_ref, q1_ref, dinv_ref):
    deg = deg_ref[0:N, 0:1] + deg_ref[N:2 * N, 0:1] + 1.0
    dinv = lax.rsqrt(deg)
    dinv_ref[...] = dinv
    p1 = jnp.dot(feats_ref[...], w1_ref[...],
                 preferred_element_type=jnp.float32)
    q1_ref[:, 0:MID] = dinv * p1
    q1_ref[:, MID:HID] = jnp.zeros((N, HID - MID), jnp.float32)


def _tc_b_body(s1_ref, q1_ref, dinv_ref, b1_ref, w2_ref, q2_ref):
    dinv = dinv_ref[...]
    q1 = q1_ref[:, 0:MID]
    s1 = s1_ref[0:N, 0:MID] + s1_ref[N:2 * N, 0:MID]
    h1 = jnp.maximum(dinv * (s1 + q1) + b1_ref[...], 0.0)
    p2 = jnp.dot(h1, w2_ref[...], preferred_element_type=jnp.float32)
    q2_ref[:, 0:MID] = dinv * p2
    q2_ref[:, MID:HID] = jnp.zeros((N, HID - MID), jnp.float32)


def _tc_c1_body(s2_ref, q2_ref, dinv_ref, b2_ref, w3_ref, b3_ref,
                wd_ref, bd_ref, h3_ref, r_ref, t_ref, tt_ref):
    dinv = dinv_ref[...]
    q2 = q2_ref[:, 0:MID]
    s2 = s2_ref[0:N, 0:MID] + s2_ref[N:2 * N, 0:MID]
    h2 = jnp.maximum(dinv * (s2 + q2) + b2_ref[...], 0.0)
    h3 = jnp.dot(h2, w3_ref[...], preferred_element_type=jnp.float32)
    h3 = h3 + b3_ref[...]
    h3_ref[...] = h3
    wd = wd_ref[...]
    g = lax.dot_general(wd, wd, (((1,), (1,)), ((), ())),
                        preferred_element_type=jnp.float32)
    bd = bd_ref[...]
    u = lax.dot_general(wd, bd, (((1,), (1,)), ((), ())),
                        preferred_element_type=jnp.float32)
    c = jnp.sum(bd * bd)
    r_ref[...] = jnp.dot(h3, g, preferred_element_type=jnp.float32)
    t = jnp.dot(h3, u, preferred_element_type=jnp.float32) + 0.5 * c
    t_ref[...] = t
    tt_ref[...] = t.reshape(1, N)


_BM = 512


def _tc_c2_body(r_ref, h3_ref, t_ref, tt_ref, o_ref):
    acc = lax.dot_general(r_ref[...], h3_ref[...], (((1,), (1,)), ((), ())),
                          preferred_element_type=jnp.float32)
    o_ref[...] = jax.nn.sigmoid(acc + t_ref[...] + tt_ref[...])


@functools.cache
def _get_tc_c2():
    return pl.pallas_call(
        _tc_c2_body,
        grid=(N // _BM, N // _BM),
        in_specs=[
            pl.BlockSpec((_BM, HID), lambda i, j: (i, 0)),
            pl.BlockSpec((_BM, HID), lambda i, j: (j, 0)),
            pl.BlockSpec((_BM, 1), lambda i, j: (i, 0)),
            pl.BlockSpec((1, _BM), lambda i, j: (0, j)),
        ],
        out_specs=pl.BlockSpec((_BM, _BM), lambda i, j: (i, j)),
        out_shape=jax.ShapeDtypeStruct((N, N), jnp.float32),
    )


def kernel(x, edge_index, table, W1, b1, W2, b2, W3, b3, Wd, bd):
    src3d = edge_index[0].reshape(NW, NCHUNK, CHUNK)
    dst3d = edge_index[1].reshape(NW, NCHUNK, CHUNK)
    ones16 = jnp.ones((CHUNK, 16), jnp.float32)
    zeros16 = jnp.zeros((CHUNK, 16), jnp.float32)
    zeros128 = jnp.zeros((CHUNK, HID), jnp.float32)

    feats, deg_parts = _get_sc_gather_deg()(table, x, dst3d, ones16, zeros16)

    q1, dinv = pl.pallas_call(
        _tc_a_body,
        out_shape=[
            jax.ShapeDtypeStruct((N, HID), jnp.float32),
            jax.ShapeDtypeStruct((N, 1), jnp.float32),
        ],
    )(deg_parts, feats, W1)

    s1_parts = _get_sc_scatter()(q1, src3d, dst3d, zeros128)

    q2 = pl.pallas_call(
        _tc_b_body,
        out_shape=jax.ShapeDtypeStruct((N, HID), jnp.float32),
    )(s1_parts, q1, dinv, b1.reshape(1, MID), W2)

    s2_parts = _get_sc_scatter()(q2, src3d, dst3d, zeros128)

    h3, r, t, tt = pl.pallas_call(
        _tc_c1_body,
        out_shape=[
            jax.ShapeDtypeStruct((N, HID), jnp.float32),
            jax.ShapeDtypeStruct((N, HID), jnp.float32),
            jax.ShapeDtypeStruct((N, 1), jnp.float32),
            jax.ShapeDtypeStruct((1, N), jnp.float32),
        ],
    )(s2_parts, q2, dinv, b2.reshape(1, MID), W3, b3.reshape(1, HID),
      Wd, bd.reshape(1, VOCAB))

    return _get_tc_c2()(r, h3, t, tt)


# speedup vs baseline: 11.2695x; 1.0211x over previous
import functools

import jax
import jax.numpy as jnp
from jax import lax
from jax.experimental import pallas as pl
from jax.experimental.pallas import tpu as pltpu
from jax.experimental.pallas import tpu_sc as plsc

N = 2048
VOCAB = 2048
HID = 128
MID = 64
E = 32768
NC, NS = 2, 16
NW = NC * NS
EPW = E // NW
CHUNK = 128
NCHUNK = EPW // CHUNK
RPW = N // NW


def _sc_gather_deg_body(table_hbm, x_hbm, dst3d_hbm, ones_hbm, zeros_hbm,
                        feats_out, deg_out,
                        xidx_v, rows_v, didx_0, didx_1, ones_v, gsem, ssem,
                        dsems, deg_sh):
    c = lax.axis_index("c")
    s = lax.axis_index("s")
    wid = s * NC + c
    dbufs = [didx_0, didx_1]

    pltpu.sync_copy(zeros_hbm, deg_sh.at[pl.ds(s * CHUNK, CHUNK)])
    pltpu.sync_copy(ones_hbm, ones_v)
    fbase = wid * RPW
    pltpu.sync_copy(x_hbm.at[pl.ds(fbase, RPW)], xidx_v)
    fg = pltpu.async_copy(table_hbm.at[xidx_v], rows_v, gsem)
    plsc.subcore_barrier()
    for i in range(NCHUNK):
        pltpu.sync_copy(dst3d_hbm.at[wid, i], dbufs[i % 2])
        pltpu.sync_copy(ones_v, deg_sh.at[dbufs[i % 2]], add=True)
    fg.wait()
    pltpu.sync_copy(rows_v, feats_out.at[pl.ds(fbase, RPW)])
    plsc.subcore_barrier()
    pltpu.sync_copy(deg_sh.at[pl.ds(s * CHUNK, CHUNK)],
                    deg_out.at[pl.ds(c * N + s * CHUNK, CHUNK)])


@functools.cache
def _get_sc_gather_deg():
    mesh = plsc.VectorSubcoreMesh(core_axis_name="c", subcore_axis_name="s",
                                  num_cores=NC, num_subcores=NS)
    return pl.kernel(
        _sc_gather_deg_body,
        out_type=[
            jax.ShapeDtypeStruct((N, HID), jnp.float32),
            jax.ShapeDtypeStruct((NC * N, 16), jnp.float32),
        ],
        mesh=mesh,
        scratch_types=[
            pltpu.VMEM((RPW,), jnp.int32),
            pltpu.VMEM((RPW, HID), jnp.float32),
            pltpu.VMEM((CHUNK,), jnp.int32),
            pltpu.VMEM((CHUNK,), jnp.int32),
            pltpu.VMEM((CHUNK, 16), jnp.float32),
            pltpu.SemaphoreType.DMA,
            pltpu.SemaphoreType.DMA,
            pltpu.SemaphoreType.DMA((2,)),
            pltpu.VMEM_SHARED((N, 16), jnp.float32),
        ],
        compiler_params=pltpu.CompilerParams(use_tc_tiling_on_sc=False),
    )


NBUF = 4


def _sc_scatter_body(q_hbm, src3d_hbm, dst3d_hbm, zeros_hbm,
                     s_out,
                     sidx_bufs, didx_bufs, row_bufs, gsems, ssems, dsems,
                     isems, agg_sh):
    c = lax.axis_index("c")
    s = lax.axis_index("s")
    wid = s * NC + c

    pltpu.sync_copy(zeros_hbm, agg_sh.at[pl.ds(s * CHUNK, CHUNK)])
    icp = [pltpu.async_copy(src3d_hbm.at[wid, i], sidx_bufs[i],
                            isems.at[i]) for i in range(NCHUNK)]
    plsc.subcore_barrier()
    gathers = [None] * NCHUNK
    for b in range(NBUF):
        icp[b].wait()
        gathers[b] = pltpu.async_copy(
            q_hbm.at[sidx_bufs[b]], row_bufs[b], gsems.at[b])
    for i in range(NCHUNK):
        b = i % NBUF
        gathers[i].wait()
        pltpu.sync_copy(dst3d_hbm.at[wid, i], didx_bufs[i % 2])
        pltpu.sync_copy(row_bufs[b], agg_sh.at[didx_bufs[i % 2]], add=True)
        if i + NBUF < NCHUNK:
            icp[i + NBUF].wait()
            gathers[i + NBUF] = pltpu.async_copy(
                q_hbm.at[sidx_bufs[i + NBUF]], row_bufs[b], gsems.at[b])
    plsc.subcore_barrier()
    pltpu.sync_copy(agg_sh.at[pl.ds(s * CHUNK, CHUNK)],
                    s_out.at[pl.ds(c * N + s * CHUNK, CHUNK)])


@functools.cache
def _get_sc_scatter():
    mesh = plsc.VectorSubcoreMesh(core_axis_name="c", subcore_axis_name="s",
                                  num_cores=NC, num_subcores=NS)
    return pl.kernel(
        _sc_scatter_body,
        out_type=jax.ShapeDtypeStruct((NC * N, MID), jnp.float32),
        mesh=mesh,
        scratch_types=[
            [pltpu.VMEM((CHUNK,), jnp.int32) for _ in range(NCHUNK)],
            [pltpu.VMEM((CHUNK,), jnp.int32) for _ in range(2)],
            [pltpu.VMEM((CHUNK, MID), jnp.float32) for _ in range(NBUF)],
            pltpu.SemaphoreType.DMA((NBUF,)),
            pltpu.SemaphoreType.DMA((NBUF,)),
            pltpu.SemaphoreType.DMA((2,)),
            pltpu.SemaphoreType.DMA((NCHUNK,)),
            pltpu.VMEM_SHARED((N, MID), jnp.float32),
        ],
        compiler_params=pltpu.CompilerParams(use_tc_tiling_on_sc=False),
    )


def _tc_a_body(deg_ref, feats_ref, w1_ref, q1_ref, dinv_ref):
    deg = deg_ref[0:N, 0:1] + deg_ref[N:2 * N, 0:1] + 1.0
    dinv = lax.rsqrt(deg)
    dinv_ref[...] = dinv
    p1 = jnp.dot(feats_ref[...], w1_ref[...],
                 preferred_element_type=jnp.float32)
    q1_ref[...] = dinv * p1


def _tc_b_body(s1_ref, q1_ref, dinv_ref, b1_ref, w2_ref, q2_ref):
    dinv = dinv_ref[...]
    q1 = q1_ref[...]
    s1 = s1_ref[0:N, :] + s1_ref[N:2 * N, :]
    h1 = jnp.maximum(dinv * (s1 + q1) + b1_ref[...], 0.0)
    p2 = jnp.dot(h1, w2_ref[...], preferred_element_type=jnp.float32)
    q2_ref[...] = dinv * p2


def _tc_c1_body(s2_ref, q2_ref, dinv_ref, b2_ref, w3_ref, b3_ref,
                wd_ref, bd_ref, h3_ref, r_ref, t_ref, tt_ref):
    dinv = dinv_ref[...]
    q2 = q2_ref[...]
    s2 = s2_ref[0:N, :] + s2_ref[N:2 * N, :]
    h2 = jnp.maximum(dinv * (s2 + q2) + b2_ref[...], 0.0)
    h3 = jnp.dot(h2, w3_ref[...], preferred_element_type=jnp.float32)
    h3 = h3 + b3_ref[...]
    h3_ref[...] = h3
    wd = wd_ref[...]
    g = lax.dot_general(wd, wd, (((1,), (1,)), ((), ())),
                        preferred_element_type=jnp.float32)
    bd = bd_ref[...]
    u = lax.dot_general(wd, bd, (((1,), (1,)), ((), ())),
                        preferred_element_type=jnp.float32)
    c = jnp.sum(bd * bd)
    r_ref[...] = jnp.dot(h3, g, preferred_element_type=jnp.float32)
    t = jnp.dot(h3, u, preferred_element_type=jnp.float32) + 0.5 * c
    t_ref[...] = t
    tt_ref[...] = t.reshape(1, N)


_BM = 512


def _tc_c2_body(r_ref, h3_ref, t_ref, tt_ref, o_ref):
    acc = lax.dot_general(r_ref[...], h3_ref[...], (((1,), (1,)), ((), ())),
                          preferred_element_type=jnp.float32)
    o_ref[...] = jax.nn.sigmoid(acc + t_ref[...] + tt_ref[...])


@functools.cache
def _get_tc_c2():
    return pl.pallas_call(
        _tc_c2_body,
        grid=(N // _BM, N // _BM),
        in_specs=[
            pl.BlockSpec((_BM, HID), lambda i, j: (i, 0)),
            pl.BlockSpec((_BM, HID), lambda i, j: (j, 0)),
            pl.BlockSpec((_BM, 1), lambda i, j: (i, 0)),
            pl.BlockSpec((1, _BM), lambda i, j: (0, j)),
        ],
        out_specs=pl.BlockSpec((_BM, _BM), lambda i, j: (i, j)),
        out_shape=jax.ShapeDtypeStruct((N, N), jnp.float32),
    )


def kernel(x, edge_index, table, W1, b1, W2, b2, W3, b3, Wd, bd):
    src3d = edge_index[0].reshape(NW, NCHUNK, CHUNK)
    dst3d = edge_index[1].reshape(NW, NCHUNK, CHUNK)
    ones16 = jnp.ones((CHUNK, 16), jnp.float32)
    zeros16 = jnp.zeros((CHUNK, 16), jnp.float32)
    zeros64 = jnp.zeros((CHUNK, MID), jnp.float32)

    feats, deg_parts = _get_sc_gather_deg()(table, x, dst3d, ones16, zeros16)

    q1, dinv = pl.pallas_call(
        _tc_a_body,
        out_shape=[
            jax.ShapeDtypeStruct((N, MID), jnp.float32),
            jax.ShapeDtypeStruct((N, 1), jnp.float32),
        ],
    )(deg_parts, feats, W1)

    s1_parts = _get_sc_scatter()(q1, src3d, dst3d, zeros64)

    q2 = pl.pallas_call(
        _tc_b_body,
        out_shape=jax.ShapeDtypeStruct((N, MID), jnp.float32),
    )(s1_parts, q1, dinv, b1.reshape(1, MID), W2)

    s2_parts = _get_sc_scatter()(q2, src3d, dst3d, zeros64)

    h3, r, t, tt = pl.pallas_call(
        _tc_c1_body,
        out_shape=[
            jax.ShapeDtypeStruct((N, HID), jnp.float32),
            jax.ShapeDtypeStruct((N, HID), jnp.float32),
            jax.ShapeDtypeStruct((N, 1), jnp.float32),
            jax.ShapeDtypeStruct((1, N), jnp.float32),
        ],
    )(s2_parts, q2, dinv, b2.reshape(1, MID), W3, b3.reshape(1, HID),
      Wd, bd.reshape(1, VOCAB))

    return _get_tc_c2()(r, h3, t, tt)


# speedup vs baseline: 11.7597x; 1.0435x over previous
import functools

import jax
import jax.numpy as jnp
from jax import lax
from jax.experimental import pallas as pl
from jax.experimental.pallas import tpu as pltpu
from jax.experimental.pallas import tpu_sc as plsc

N = 2048
VOCAB = 2048
HID = 128
MID = 64
E = 32768
NC, NS = 2, 16
NW = NC * NS
EPW = E // NW
CHUNK = 128
NCHUNK = EPW // CHUNK
RPW = N // NW


def _sc_gather_deg_body(table_hbm, x_hbm, dst3d_hbm, ones_hbm, zeros_hbm,
                        feats_out, deg_out,
                        xidx_v, rows_v, didx_0, didx_1, ones_v, gsem, ssem,
                        dsems, deg_sh):
    c = lax.axis_index("c")
    s = lax.axis_index("s")
    wid = s * NC + c
    dbufs = [didx_0, didx_1]

    pltpu.sync_copy(zeros_hbm, deg_sh.at[pl.ds(s * CHUNK, CHUNK)])
    pltpu.sync_copy(ones_hbm, ones_v)
    fbase = wid * RPW
    pltpu.sync_copy(x_hbm.at[pl.ds(fbase, RPW)], xidx_v)
    fg = pltpu.async_copy(table_hbm.at[xidx_v], rows_v, gsem)
    plsc.subcore_barrier()
    for i in range(NCHUNK):
        pltpu.sync_copy(dst3d_hbm.at[wid, i], dbufs[i % 2])
        pltpu.sync_copy(ones_v, deg_sh.at[dbufs[i % 2]], add=True)
    fg.wait()
    pltpu.sync_copy(rows_v, feats_out.at[pl.ds(fbase, RPW)])
    plsc.subcore_barrier()
    pltpu.sync_copy(deg_sh.at[pl.ds(s * CHUNK, CHUNK)],
                    deg_out.at[pl.ds(c * N + s * CHUNK, CHUNK)])


@functools.cache
def _get_sc_gather_deg():
    mesh = plsc.VectorSubcoreMesh(core_axis_name="c", subcore_axis_name="s",
                                  num_cores=NC, num_subcores=NS)
    return pl.kernel(
        _sc_gather_deg_body,
        out_type=[
            jax.ShapeDtypeStruct((N, HID), jnp.float32),
            jax.ShapeDtypeStruct((NC * N, 16), jnp.float32),
        ],
        mesh=mesh,
        scratch_types=[
            pltpu.VMEM((RPW,), jnp.int32),
            pltpu.VMEM((RPW, HID), jnp.float32),
            pltpu.VMEM((CHUNK,), jnp.int32),
            pltpu.VMEM((CHUNK,), jnp.int32),
            pltpu.VMEM((CHUNK, 16), jnp.float32),
            pltpu.SemaphoreType.DMA,
            pltpu.SemaphoreType.DMA,
            pltpu.SemaphoreType.DMA((2,)),
            pltpu.VMEM_SHARED((N, 16), jnp.float32),
        ],
        compiler_params=pltpu.CompilerParams(use_tc_tiling_on_sc=False),
    )


NBUF = 4


def _sc_scatter_body(q_hbm, src3d_hbm, dst3d_hbm, zeros_hbm,
                     s_out,
                     sidx_bufs, didx_bufs, row_bufs, gsems, ssems, dsems,
                     isems, agg_sh):
    c = lax.axis_index("c")
    s = lax.axis_index("s")
    wid = s * NC + c

    pltpu.sync_copy(zeros_hbm, agg_sh.at[pl.ds(s * CHUNK, CHUNK)])
    icp = [pltpu.async_copy(src3d_hbm.at[wid, i], sidx_bufs[i],
                            isems.at[i]) for i in range(NCHUNK)]
    plsc.subcore_barrier()
    gathers = [None] * NCHUNK
    for b in range(NBUF):
        icp[b].wait()
        gathers[b] = pltpu.async_copy(
            q_hbm.at[sidx_bufs[b]], row_bufs[b], gsems.at[b])
    for i in range(NCHUNK):
        b = i % NBUF
        gathers[i].wait()
        pltpu.sync_copy(dst3d_hbm.at[wid, i], didx_bufs[i % 2])
        pltpu.sync_copy(row_bufs[b], agg_sh.at[didx_bufs[i % 2]], add=True)
        if i + NBUF < NCHUNK:
            icp[i + NBUF].wait()
            gathers[i + NBUF] = pltpu.async_copy(
                q_hbm.at[sidx_bufs[i + NBUF]], row_bufs[b], gsems.at[b])
    plsc.subcore_barrier()
    pltpu.sync_copy(agg_sh.at[pl.ds(s * CHUNK, CHUNK)],
                    s_out.at[pl.ds(c * N + s * CHUNK, CHUNK)])


@functools.cache
def _get_sc_scatter():
    mesh = plsc.VectorSubcoreMesh(core_axis_name="c", subcore_axis_name="s",
                                  num_cores=NC, num_subcores=NS)
    return pl.kernel(
        _sc_scatter_body,
        out_type=jax.ShapeDtypeStruct((NC * N, MID), jnp.float32),
        mesh=mesh,
        scratch_types=[
            [pltpu.VMEM((CHUNK,), jnp.int32) for _ in range(NCHUNK)],
            [pltpu.VMEM((CHUNK,), jnp.int32) for _ in range(2)],
            [pltpu.VMEM((CHUNK, MID), jnp.float32) for _ in range(NBUF)],
            pltpu.SemaphoreType.DMA((NBUF,)),
            pltpu.SemaphoreType.DMA((NBUF,)),
            pltpu.SemaphoreType.DMA((2,)),
            pltpu.SemaphoreType.DMA((NCHUNK,)),
            pltpu.VMEM_SHARED((N, MID), jnp.float32),
        ],
        compiler_params=pltpu.CompilerParams(use_tc_tiling_on_sc=False),
    )


def _tc_a_body(deg_ref, feats_ref, w1_ref, q1_ref, dinv_ref):
    deg = deg_ref[0:N, 0:1] + deg_ref[N:2 * N, 0:1] + 1.0
    dinv = lax.rsqrt(deg)
    dinv_ref[...] = dinv
    p1 = jnp.dot(feats_ref[...], w1_ref[...],
                 preferred_element_type=jnp.float32)
    q1_ref[...] = dinv * p1


def _tc_b_body(s1_ref, q1_ref, dinv_ref, b1_ref, w2_ref, q2_ref):
    dinv = dinv_ref[...]
    q1 = q1_ref[...]
    s1 = s1_ref[0:N, :] + s1_ref[N:2 * N, :]
    h1 = jnp.maximum(dinv * (s1 + q1) + b1_ref[...], 0.0)
    p2 = jnp.dot(h1, w2_ref[...], preferred_element_type=jnp.float32)
    q2_ref[...] = dinv * p2


_BM = 512


def _tc_c_body(s2_ref, q2_ref, dinv_ref, b2_ref, w3_ref, b3_ref,
               wd_ref, bd_ref, o_ref, h3_s, r_s, t_s):
    i = pl.program_id(0)
    j = pl.program_id(1)

    @pl.when(jnp.logical_and(i == 0, j == 0))
    def _prologue():
        dinv = dinv_ref[...]
        q2 = q2_ref[...]
        s2 = s2_ref[0:N, :] + s2_ref[N:2 * N, :]
        h2 = jnp.maximum(dinv * (s2 + q2) + b2_ref[...], 0.0)
        h3 = jnp.dot(h2, w3_ref[...], preferred_element_type=jnp.float32)
        h3 = h3 + b3_ref[...]
        h3_s[...] = h3
        wd = wd_ref[...]
        g = lax.dot_general(wd, wd, (((1,), (1,)), ((), ())),
                            preferred_element_type=jnp.float32)
        bd = bd_ref[...]
        u = lax.dot_general(wd, bd, (((1,), (1,)), ((), ())),
                            preferred_element_type=jnp.float32)
        c = jnp.sum(bd * bd)
        r_s[...] = jnp.dot(h3, g, preferred_element_type=jnp.float32)
        t_s[...] = (jnp.dot(h3, u, preferred_element_type=jnp.float32)
                    + 0.5 * c)

    acc = lax.dot_general(r_s[pl.ds(i * _BM, _BM), :],
                          h3_s[pl.ds(j * _BM, _BM), :],
                          (((1,), (1,)), ((), ())),
                          preferred_element_type=jnp.float32)
    ti = t_s[pl.ds(i * _BM, _BM), :]
    tj = t_s[pl.ds(j * _BM, _BM), :].reshape(1, _BM)
    o_ref[...] = jax.nn.sigmoid(acc + ti + tj)


@functools.cache
def _get_tc_c():
    full = lambda i, j: (0, 0)
    return pl.pallas_call(
        _tc_c_body,
        grid=(N // _BM, N // _BM),
        in_specs=[
            pl.BlockSpec((NC * N, MID), full),
            pl.BlockSpec((N, MID), full),
            pl.BlockSpec((N, 1), full),
            pl.BlockSpec((1, MID), full),
            pl.BlockSpec((MID, HID), full),
            pl.BlockSpec((1, HID), full),
            pl.BlockSpec((HID, VOCAB), full),
            pl.BlockSpec((1, VOCAB), full),
        ],
        out_specs=pl.BlockSpec((_BM, _BM), lambda i, j: (i, j)),
        out_shape=jax.ShapeDtypeStruct((N, N), jnp.float32),
        scratch_shapes=[
            pltpu.VMEM((N, HID), jnp.float32),
            pltpu.VMEM((N, HID), jnp.float32),
            pltpu.VMEM((N, 1), jnp.float32),
        ],
    )


def kernel(x, edge_index, table, W1, b1, W2, b2, W3, b3, Wd, bd):
    src3d = edge_index[0].reshape(NW, NCHUNK, CHUNK)
    dst3d = edge_index[1].reshape(NW, NCHUNK, CHUNK)
    ones16 = jnp.ones((CHUNK, 16), jnp.float32)
    zeros16 = jnp.zeros((CHUNK, 16), jnp.float32)
    zeros64 = jnp.zeros((CHUNK, MID), jnp.float32)

    feats, deg_parts = _get_sc_gather_deg()(table, x, dst3d, ones16, zeros16)

    q1, dinv = pl.pallas_call(
        _tc_a_body,
        out_shape=[
            jax.ShapeDtypeStruct((N, MID), jnp.float32),
            jax.ShapeDtypeStruct((N, 1), jnp.float32),
        ],
    )(deg_parts, feats, W1)

    s1_parts = _get_sc_scatter()(q1, src3d, dst3d, zeros64)

    q2 = pl.pallas_call(
        _tc_b_body,
        out_shape=jax.ShapeDtypeStruct((N, MID), jnp.float32),
    )(s1_parts, q1, dinv, b1.reshape(1, MID), W2)

    s2_parts = _get_sc_scatter()(q2, src3d, dst3d, zeros64)

    return _get_tc_c()(s2_parts, q2, dinv, b2.reshape(1, MID), W3,
                       b3.reshape(1, HID), Wd, bd.reshape(1, VOCAB))
